# baseline (speedup 1.0000x reference)
import jax
import jax.numpy as jnp
from jax.experimental import pallas as pl
from jax.experimental.pallas import tpu as pltpu
from jax.experimental.pallas import tpu_sc as plsc

B = 4096
L = 50
EMB = 128

BLOCK_B = 8
CHUNKS = 4
CHUNK_B = B // CHUNKS
TC_ROWS = 8


def _sc_gather_chunk(table, x, chunk):
    mesh = plsc.VectorSubcoreMesh(core_axis_name="core",
                                  subcore_axis_name="subcore")
    base = chunk * (CHUNK_B // BLOCK_B)

    @pl.kernel(out_type=jax.ShapeDtypeStruct((CHUNK_B * L, EMB), table.dtype),
               mesh=mesh,
               scratch_types=[pltpu.SemaphoreType.DMA])
    def gather_kernel(table_hbm, i_hbm, o_hbm, sem):
        def body(i_vmem, o_vmem):
            copies = [
                pltpu.make_async_copy(table_hbm.at[i_vmem.at[b]],
                                      o_vmem.at[pl.ds(b * L, L)], sem)
                for b in range(BLOCK_B)
            ]
            for c in copies:
                c.start()
            for c in copies:
                c.wait()

        pltpu.emit_pipeline(
            body,
            grid=(CHUNK_B // BLOCK_B,),
            in_specs=[pl.BlockSpec((BLOCK_B, L),
                                   index_map=lambda i: (base + i, 0))],
            out_specs=[pl.BlockSpec((BLOCK_B * L, EMB),
                                    index_map=lambda i: (i, 0))],
            core_axis_name=("core", "subcore"),
            dimension_semantics=(pltpu.PARALLEL,),
        )(i_hbm, o_hbm)

    return gather_kernel(table, x)


def _tc_scatter_first(flat):
    def body(flat_ref, o_ref):
        o_ref[...] = flat_ref[...].reshape(TC_ROWS, L, EMB)

    return pl.pallas_call(
        body,
        grid=(CHUNK_B // TC_ROWS,),
        in_specs=[pl.BlockSpec((TC_ROWS * L, EMB), lambda i: (i, 0))],
        out_specs=pl.BlockSpec((TC_ROWS, L, EMB), lambda i: (i, 0, 0)),
        out_shape=jax.ShapeDtypeStruct((B, L, EMB), flat.dtype),
    )(flat)


def _tc_scatter_chunk(big, flat, chunk):
    base = chunk * (CHUNK_B // TC_ROWS)

    def body(big_ref, flat_ref, o_ref):
        del big_ref
        o_ref[...] = flat_ref[...].reshape(TC_ROWS, L, EMB)

    return pl.pallas_call(
        body,
        grid=(CHUNK_B // TC_ROWS,),
        in_specs=[pl.BlockSpec(memory_space=pl.ANY),
                  pl.BlockSpec((TC_ROWS * L, EMB), lambda i: (i, 0))],
        out_specs=pl.BlockSpec((TC_ROWS, L, EMB),
                               lambda i: (base + i, 0, 0)),
        out_shape=jax.ShapeDtypeStruct((B, L, EMB), flat.dtype),
        input_output_aliases={0: 0},
    )(big, flat)


def kernel(x, table):
    xi = x.astype(jnp.int32)
    flats = [_sc_gather_chunk(table, xi, k) for k in range(CHUNKS)]
    big = _tc_scatter_first(flats[0])
    for k in range(1, CHUNKS):
        big = _tc_scatter_chunk(big, flats[k], k)
    return big


# speedup vs baseline: 1.8172x; 1.8172x over previous
import jax
import jax.numpy as jnp
from jax.experimental import pallas as pl
from jax.experimental.pallas import tpu as pltpu
from jax.experimental.pallas import tpu_sc as plsc

B = 4096
L = 50
LPAD = 56
EMB = 128

BLOCK_B = 8
CHUNKS = 4
CHUNK_B = B // CHUNKS
TC_ROWS = 64


def _sc_gather_chunk(table, x, chunk):
    mesh = plsc.VectorSubcoreMesh(core_axis_name="core",
                                  subcore_axis_name="subcore")
    base = chunk * (CHUNK_B // BLOCK_B)

    @pl.kernel(out_type=jax.ShapeDtypeStruct((CHUNK_B * LPAD, EMB),
                                             table.dtype),
               mesh=mesh,
               scratch_types=[pltpu.SemaphoreType.DMA])
    def gather_kernel(table_hbm, i_hbm, o_hbm, sem):
        def body(i_vmem, o_vmem):
            copies = [
                pltpu.make_async_copy(table_hbm.at[i_vmem.at[b]],
                                      o_vmem.at[pl.ds(b * LPAD, L)], sem)
                for b in range(BLOCK_B)
            ]
            for c in copies:
                c.start()
            for c in copies:
                c.wait()

        pltpu.emit_pipeline(
            body,
            grid=(CHUNK_B // BLOCK_B,),
            in_specs=[pl.BlockSpec((BLOCK_B, L),
                                   index_map=lambda i: (base + i, 0))],
            out_specs=[pl.BlockSpec((BLOCK_B * LPAD, EMB),
                                    index_map=lambda i: (i, 0))],
            core_axis_name=("core", "subcore"),
            dimension_semantics=(pltpu.PARALLEL,),
        )(i_hbm, o_hbm)

    return gather_kernel(table, x)


def _tc_relayout_body(flat_ref, o_ref):
    o_ref[...] = flat_ref[...].reshape(TC_ROWS, LPAD, EMB)[:, :L, :]


def _tc_scatter_first(flat):
    def body(flat_ref, o_ref):
        _tc_relayout_body(flat_ref, o_ref)

    return pl.pallas_call(
        body,
        grid=(CHUNK_B // TC_ROWS,),
        in_specs=[pl.BlockSpec((TC_ROWS * LPAD, EMB), lambda i: (i, 0))],
        out_specs=pl.BlockSpec((TC_ROWS, L, EMB), lambda i: (i, 0, 0)),
        out_shape=jax.ShapeDtypeStruct((B, L, EMB), flat.dtype),
    )(flat)


def _tc_scatter_chunk(big, flat, chunk):
    base = chunk * (CHUNK_B // TC_ROWS)

    def body(big_ref, flat_ref, o_ref):
        del big_ref
        _tc_relayout_body(flat_ref, o_ref)

    return pl.pallas_call(
        body,
        grid=(CHUNK_B // TC_ROWS,),
        in_specs=[pl.BlockSpec(memory_space=pl.ANY),
                  pl.BlockSpec((TC_ROWS * LPAD, EMB), lambda i: (i, 0))],
        out_specs=pl.BlockSpec((TC_ROWS, L, EMB),
                               lambda i: (base + i, 0, 0)),
        out_shape=jax.ShapeDtypeStruct((B, L, EMB), flat.dtype),
        input_output_aliases={0: 0},
    )(big, flat)


def kernel(x, table):
    xi = x.astype(jnp.int32)
    flats = [_sc_gather_chunk(table, xi, k) for k in range(CHUNKS)]
    big = _tc_scatter_first(flats[0])
    for k in range(1, CHUNKS):
        big = _tc_scatter_chunk(big, flats[k], k)
    return big


# speedup vs baseline: 4.3257x; 2.3804x over previous
import jax
import jax.numpy as jnp
from jax import lax
from jax.experimental import pallas as pl
from jax.experimental.pallas import tpu as pltpu
from jax.experimental.pallas import tpu_sc as plsc

B = 4096
L = 50
EMB = 128

NC = 2
NS = 16
NW = NC * NS
COLS = B // NW


def _sc_gather_lmajor(table, idx_t):
    mesh = plsc.VectorSubcoreMesh(core_axis_name="core",
                                  subcore_axis_name="subcore")

    @pl.kernel(out_type=jax.ShapeDtypeStruct((L, B, EMB), table.dtype),
               mesh=mesh,
               scratch_types=[
                   pltpu.VMEM((L, COLS), jnp.int32),
                   pltpu.VMEM((COLS, EMB), jnp.float32),
                   pltpu.VMEM((COLS, EMB), jnp.float32),
                   pltpu.SemaphoreType.DMA,
                   pltpu.SemaphoreType.DMA,
                   pltpu.SemaphoreType.DMA,
                   pltpu.SemaphoreType.DMA,
                   pltpu.SemaphoreType.DMA,
               ])
    def gather_kernel(table_hbm, i_hbm, o_hbm, idx_v, buf0, buf1,
                      si, sg0, sg1, so0, so1):
        core = lax.axis_index("core")
        sub = lax.axis_index("subcore")
        base = (sub * NC + core) * COLS

        pltpu.async_copy(i_hbm.at[:, pl.ds(base, COLS)], idx_v, si).wait()

        @pl.loop(0, L // 2)
        def _(i):
            l0 = 2 * i
            g0 = pltpu.make_async_copy(table_hbm.at[idx_v.at[l0]], buf0, sg0)
            g1 = pltpu.make_async_copy(table_hbm.at[idx_v.at[l0 + 1]],
                                       buf1, sg1)
            g0.start()
            g1.start()
            g0.wait()
            o0 = pltpu.make_async_copy(buf0,
                                       o_hbm.at[l0, pl.ds(base, COLS)], so0)
            o0.start()
            g1.wait()
            o1 = pltpu.make_async_copy(buf1,
                                       o_hbm.at[l0 + 1, pl.ds(base, COLS)],
                                       so1)
            o1.start()
            o0.wait()
            o1.wait()

    return gather_kernel(table, idx_t)


def kernel(x, table):
    idx_t = x.astype(jnp.int32).T
    out_t = _sc_gather_lmajor(table, idx_t)
    return jnp.transpose(out_t, (1, 0, 2))


# speedup vs baseline: 4.8849x; 1.1293x over previous
import jax
import jax.numpy as jnp
from jax import lax
from jax.experimental import pallas as pl
from jax.experimental.pallas import tpu as pltpu
from jax.experimental.pallas import tpu_sc as plsc

B = 4096
L = 50
EMB = 128

NC = 2
NS = 16
NW = NC * NS
COLS = B // NW


def _sc_gather_lmajor(table, idx_t):
    mesh = plsc.VectorSubcoreMesh(core_axis_name="core",
                                  subcore_axis_name="subcore")

    @pl.kernel(out_type=jax.ShapeDtypeStruct((L, B, EMB), table.dtype),
               mesh=mesh,
               scratch_types=[
                   pltpu.VMEM((L, COLS), jnp.int32),
                   pltpu.VMEM((COLS, EMB), jnp.float32),
                   pltpu.VMEM((COLS, EMB), jnp.float32),
                   pltpu.VMEM((COLS, EMB), jnp.float32),
                   pltpu.VMEM((COLS, EMB), jnp.float32),
                   pltpu.SemaphoreType.DMA,
                   pltpu.SemaphoreType.DMA,
                   pltpu.SemaphoreType.DMA,
                   pltpu.SemaphoreType.DMA,
                   pltpu.SemaphoreType.DMA,
                   pltpu.SemaphoreType.DMA,
                   pltpu.SemaphoreType.DMA,
                   pltpu.SemaphoreType.DMA,
                   pltpu.SemaphoreType.DMA,
               ])
    def gather_kernel(table_hbm, i_hbm, o_hbm, idx_v,
                      bufa0, bufa1, bufb0, bufb1,
                      si, sga0, sga1, sgb0, sgb1, soa0, soa1, sob0, sob1):
        core = lax.axis_index("core")
        sub = lax.axis_index("subcore")
        base = (sub * NC + core) * COLS

        pltpu.async_copy(i_hbm.at[:, pl.ds(base, COLS)], idx_v, si).wait()

        def gth(l, buf, sem):
            return pltpu.make_async_copy(table_hbm.at[idx_v.at[l]], buf, sem)

        def out(l, buf, sem):
            return pltpu.make_async_copy(buf, o_hbm.at[l, pl.ds(base, COLS)],
                                         sem)

        gth(0, bufa0, sga0).start()
        gth(1, bufa1, sga1).start()
        gth(2, bufb0, sgb0).start()
        gth(3, bufb1, sgb1).start()
        gth(0, bufa0, sga0).wait()
        out(0, bufa0, soa0).start()
        gth(1, bufa1, sga1).wait()
        out(1, bufa1, soa1).start()

        @pl.loop(0, 11)
        def _(k):
            l = 4 * k
            gth(l + 2, bufb0, sgb0).wait()
            out(l + 2, bufb0, sob0).start()
            gth(l + 3, bufb1, sgb1).wait()
            out(l + 3, bufb1, sob1).start()
            out(l, bufa0, soa0).wait()
            gth(l + 4, bufa0, sga0).start()
            out(l + 1, bufa1, soa1).wait()
            gth(l + 5, bufa1, sga1).start()
            gth(l + 4, bufa0, sga0).wait()
            out(l + 4, bufa0, soa0).start()
            gth(l + 5, bufa1, sga1).wait()
            out(l + 5, bufa1, soa1).start()
            out(l + 2, bufb0, sob0).wait()
            gth(l + 6, bufb0, sgb0).start()
            out(l + 3, bufb1, sob1).wait()
            gth(l + 7, bufb1, sgb1).start()

        gth(46, bufb0, sgb0).wait()
        out(46, bufb0, sob0).start()
        gth(47, bufb1, sgb1).wait()
        out(47, bufb1, sob1).start()
        out(44, bufa0, soa0).wait()
        gth(48, bufa0, sga0).start()
        out(45, bufa1, soa1).wait()
        gth(49, bufa1, sga1).start()
        gth(48, bufa0, sga0).wait()
        out(48, bufa0, soa0).start()
        gth(49, bufa1, sga1).wait()
        out(49, bufa1, soa1).start()
        out(46, bufb0, sob0).wait()
        out(47, bufb1, sob1).wait()
        out(48, bufa0, soa0).wait()
        out(49, bufa1, soa1).wait()

    return gather_kernel(table, idx_t)


def kernel(x, table):
    idx_t = x.astype(jnp.int32).T
    out_t = _sc_gather_lmajor(table, idx_t)
    return jnp.transpose(out_t, (1, 0, 2))
